# Initial kernel scaffold; baseline (speedup 1.0000x reference)
import jax
import jax.numpy as jnp
from jax.experimental import pallas as pl


def kernel(x, norm_scale, gate_w, gate_b, mlp1_weight, mlp1_bias, mlp2_weight, mlp2_bias):
    raise NotImplementedError("write your pallas kernel here")


# speedup vs baseline: 7.0336x; 7.0336x over previous
import functools

import jax
import jax.numpy as jnp
from jax.experimental import pallas as pl
from jax.experimental.pallas import tpu as pltpu

E = 64
K = 2
H = 768
I = 768
T = 128
ALPHA = 1.702
LIMIT = 7.0
W_LOAD = 0.01
W_IMP = 0.01
W_AUX = 1.0
EPS = 1e-06


def _moe_kernel(x_ref, scale_ref, gw_ref, gb_ref, w1_ref, b1_ref,
                w2_ref, b2_ref, out_ref, aux_ref, t_ref, wfull_ref, pair_ref):
    e = pl.program_id(0)

    @pl.when(e == 0)
    def _prologue():
        xx = x_ref[...]
        ms = jnp.mean(xx * xx, axis=1, keepdims=True)
        t = xx * jax.lax.rsqrt(ms + EPS)
        t = t * scale_ref[...]
        t_ref[...] = t
        logits = jax.lax.dot_general(
            t, gw_ref[...], (((1,), (1,)), ((), ())),
            preferred_element_type=jnp.float32) + gb_ref[...]
        lmax = jnp.max(logits, axis=1, keepdims=True)
        ex = jnp.exp(logits - lmax)
        probs = ex / jnp.sum(ex, axis=1, keepdims=True)
        eids = jax.lax.broadcasted_iota(jnp.int32, (T, E), 1)
        m1 = jnp.max(probs, axis=1, keepdims=True)
        a1 = jnp.min(jnp.where(probs == m1, eids, E), axis=1, keepdims=True)
        oh1 = (eids == a1)
        probs2 = jnp.where(oh1, -1.0, probs)
        m2 = jnp.max(probs2, axis=1, keepdims=True)
        a2 = jnp.min(jnp.where(probs2 == m2, eids, E), axis=1, keepdims=True)
        oh2 = (eids == a2)
        denom = m1 + m2
        wfull_ref[...] = (jnp.where(oh1, m1, 0.0) +
                          jnp.where(oh2, m2, 0.0)) / denom
        p_mean = jnp.mean(probs, axis=0)
        c1 = jnp.sum(oh1.astype(jnp.float32), axis=0)
        c2 = jnp.sum(oh2.astype(jnp.float32), axis=0)
        d = (4.0 * c1 + 2.0 * c2) / (T * K)
        loss_lb = W_LOAD * E * jnp.sum(p_mean * d)
        imp = jnp.sum(logits, axis=0)
        imp_mean = jnp.mean(imp)
        var = jnp.sum((imp - imp_mean) ** 2) / (E - 1)
        cv = jnp.sqrt(var) / (imp_mean + 1e-06)
        aux = W_AUX * (loss_lb + W_IMP * cv * cv)
        aux_ref[...] = jnp.reshape(aux, (1, 1))
        out_ref[...] = xx
        j_iota = jax.lax.broadcasted_iota(jnp.int32, (2 * I, I), 0)
        k_iota = jax.lax.broadcasted_iota(jnp.int32, (2 * I, I), 1)
        pair_ref[...] = ((j_iota // 2) == k_iota).astype(jnp.bfloat16)

    tb = t_ref[...].astype(jnp.bfloat16)
    w1 = w1_ref[0].astype(jnp.bfloat16)
    h = jax.lax.dot_general(tb, w1, (((1,), (1,)), ((), ())),
                            preferred_element_type=jnp.float32)
    h = jnp.clip(h + b1_ref[0], -LIMIT, LIMIT)
    lane = jax.lax.broadcasted_iota(jnp.int32, (T, 2 * I), 1)
    even = (lane % 2) == 0
    gi = jnp.where(even, h * jax.nn.sigmoid(h * ALPHA), h + 1.0)
    g = jax.lax.dot_general(gi.astype(jnp.bfloat16), pair_ref[...],
                            (((1,), (0,)), ((), ())),
                            preferred_element_type=jnp.float32)
    gb16 = g.astype(jnp.bfloat16)
    w2 = w2_ref[0].astype(jnp.bfloat16)
    o = jax.lax.dot_general(gb16, w2, (((1,), (1,)), ((), ())),
                            preferred_element_type=jnp.float32)
    o = o + b2_ref[0]
    eids = jax.lax.broadcasted_iota(jnp.int32, (T, E), 1)
    wcol = jnp.sum(jnp.where(eids == e, wfull_ref[...], 0.0), axis=1,
                   keepdims=True)
    out_ref[...] += wcol * o


@functools.partial(jax.jit)
def kernel(x, norm_scale, gate_w, gate_b, mlp1_weight, mlp1_bias,
           mlp2_weight, mlp2_bias):
    out, aux = pl.pallas_call(
        _moe_kernel,
        grid=(E,),
        in_specs=[
            pl.BlockSpec((T, H), lambda e: (0, 0)),
            pl.BlockSpec((1, H), lambda e: (0, 0)),
            pl.BlockSpec((E, H), lambda e: (0, 0)),
            pl.BlockSpec((1, E), lambda e: (0, 0)),
            pl.BlockSpec((1, 2 * I, H), lambda e: (e, 0, 0)),
            pl.BlockSpec((1, 1, 2 * I), lambda e: (e, 0, 0)),
            pl.BlockSpec((1, H, I), lambda e: (e, 0, 0)),
            pl.BlockSpec((1, 1, H), lambda e: (e, 0, 0)),
        ],
        out_specs=[
            pl.BlockSpec((T, H), lambda e: (0, 0)),
            pl.BlockSpec((1, 1), lambda e: (0, 0)),
        ],
        out_shape=[
            jax.ShapeDtypeStruct((T, H), jnp.float32),
            jax.ShapeDtypeStruct((1, 1), jnp.float32),
        ],
        scratch_shapes=[
            pltpu.VMEM((T, H), jnp.float32),
            pltpu.VMEM((T, E), jnp.float32),
            pltpu.VMEM((2 * I, I), jnp.bfloat16),
        ],
        compiler_params=pltpu.CompilerParams(
            dimension_semantics=("arbitrary",)),
    )(x, jnp.reshape(norm_scale, (1, H)), gate_w, jnp.reshape(gate_b, (1, E)),
      mlp1_weight, jnp.reshape(mlp1_bias, (E, 1, 2 * I)),
      mlp2_weight, jnp.reshape(mlp2_bias, (E, 1, H)))
    return out, jnp.reshape(aux, ())


# speedup vs baseline: 8.1085x; 1.1528x over previous
import functools

import jax
import jax.numpy as jnp
from jax.experimental import pallas as pl
from jax.experimental.pallas import tpu as pltpu

E = 64
K = 2
H = 768
I = 768
T = 128
ALPHA = 1.702
LIMIT = 7.0
W_LOAD = 0.01
W_IMP = 0.01
W_AUX = 1.0
EPS = 1e-06


EB = 2


def _moe_kernel(x_ref, scale_ref, gw_ref, gb_ref, w1_ref, b1_ref,
                w2_ref, b2_ref, out_ref, aux_ref, t_ref, wfull_ref, pair_ref):
    i = pl.program_id(0)

    @pl.when(i == 0)
    def _prologue():
        xx = x_ref[...]
        ms = jnp.mean(xx * xx, axis=1, keepdims=True)
        t = xx * jax.lax.rsqrt(ms + EPS)
        t = t * scale_ref[...]
        t_ref[...] = t
        logits = jax.lax.dot_general(
            t, gw_ref[...], (((1,), (1,)), ((), ())),
            preferred_element_type=jnp.float32) + gb_ref[...]
        lmax = jnp.max(logits, axis=1, keepdims=True)
        ex = jnp.exp(logits - lmax)
        probs = ex / jnp.sum(ex, axis=1, keepdims=True)
        eids = jax.lax.broadcasted_iota(jnp.int32, (T, E), 1)
        m1 = jnp.max(probs, axis=1, keepdims=True)
        a1 = jnp.min(jnp.where(probs == m1, eids, E), axis=1, keepdims=True)
        oh1 = (eids == a1)
        probs2 = jnp.where(oh1, -1.0, probs)
        m2 = jnp.max(probs2, axis=1, keepdims=True)
        a2 = jnp.min(jnp.where(probs2 == m2, eids, E), axis=1, keepdims=True)
        oh2 = (eids == a2)
        denom = m1 + m2
        wfull_ref[...] = (jnp.where(oh1, m1, 0.0) +
                          jnp.where(oh2, m2, 0.0)) / denom
        p_mean = jnp.mean(probs, axis=0)
        c1 = jnp.sum(oh1.astype(jnp.float32), axis=0)
        c2 = jnp.sum(oh2.astype(jnp.float32), axis=0)
        d = (4.0 * c1 + 2.0 * c2) / (T * K)
        loss_lb = W_LOAD * E * jnp.sum(p_mean * d)
        imp = jnp.sum(logits, axis=0)
        imp_mean = jnp.mean(imp)
        var = jnp.sum((imp - imp_mean) ** 2) / (E - 1)
        cv = jnp.sqrt(var) / (imp_mean + 1e-06)
        aux = W_AUX * (loss_lb + W_IMP * cv * cv)
        aux_ref[...] = jnp.reshape(aux, (1, 1))
        out_ref[...] = xx
        j_iota = jax.lax.broadcasted_iota(jnp.int32, (2 * I, I), 0)
        k_iota = jax.lax.broadcasted_iota(jnp.int32, (2 * I, I), 1)
        pair_ref[...] = ((j_iota // 2) == k_iota).astype(jnp.bfloat16)

    tb = t_ref[...].astype(jnp.bfloat16)
    eids = jax.lax.broadcasted_iota(jnp.int32, (T, E), 1)
    acc = out_ref[...]
    for p in range(EB):
        e = EB * i + p
        w1 = w1_ref[p].astype(jnp.bfloat16)
        h = jax.lax.dot_general(tb, w1, (((1,), (1,)), ((), ())),
                                preferred_element_type=jnp.float32)
        h = jnp.clip(h + b1_ref[p], -LIMIT, LIMIT)
        lane = jax.lax.broadcasted_iota(jnp.int32, (T, 2 * I), 1)
        evn = (lane % 2) == 0
        gi = jnp.where(evn, h * jax.nn.sigmoid(h * ALPHA), h + 1.0)
        g = jax.lax.dot_general(gi.astype(jnp.bfloat16), pair_ref[...],
                                (((1,), (0,)), ((), ())),
                                preferred_element_type=jnp.float32)
        gb16 = g.astype(jnp.bfloat16)
        w2 = w2_ref[p].astype(jnp.bfloat16)
        o = jax.lax.dot_general(gb16, w2, (((1,), (1,)), ((), ())),
                                preferred_element_type=jnp.float32)
        o = o + b2_ref[p]
        wcol = jnp.sum(jnp.where(eids == e, wfull_ref[...], 0.0), axis=1,
                       keepdims=True)
        acc = acc + wcol * o
    out_ref[...] = acc


@functools.partial(jax.jit)
def kernel(x, norm_scale, gate_w, gate_b, mlp1_weight, mlp1_bias,
           mlp2_weight, mlp2_bias):
    out, aux = pl.pallas_call(
        _moe_kernel,
        grid=(E // EB,),
        in_specs=[
            pl.BlockSpec((T, H), lambda e: (0, 0)),
            pl.BlockSpec((1, H), lambda e: (0, 0)),
            pl.BlockSpec((E, H), lambda e: (0, 0)),
            pl.BlockSpec((1, E), lambda e: (0, 0)),
            pl.BlockSpec((EB, 2 * I, H), lambda e: (e, 0, 0)),
            pl.BlockSpec((EB, 1, 2 * I), lambda e: (e, 0, 0)),
            pl.BlockSpec((EB, H, I), lambda e: (e, 0, 0)),
            pl.BlockSpec((EB, 1, H), lambda e: (e, 0, 0)),
        ],
        out_specs=[
            pl.BlockSpec((T, H), lambda e: (0, 0)),
            pl.BlockSpec((1, 1), lambda e: (0, 0)),
        ],
        out_shape=[
            jax.ShapeDtypeStruct((T, H), jnp.float32),
            jax.ShapeDtypeStruct((1, 1), jnp.float32),
        ],
        scratch_shapes=[
            pltpu.VMEM((T, H), jnp.float32),
            pltpu.VMEM((T, E), jnp.float32),
            pltpu.VMEM((2 * I, I), jnp.bfloat16),
        ],
        compiler_params=pltpu.CompilerParams(
            dimension_semantics=("arbitrary",)),
    )(x, jnp.reshape(norm_scale, (1, H)), gate_w, jnp.reshape(gate_b, (1, E)),
      mlp1_weight, jnp.reshape(mlp1_bias, (E, 1, 2 * I)),
      mlp2_weight, jnp.reshape(mlp2_bias, (E, 1, H)))
    return out, jnp.reshape(aux, ())


# speedup vs baseline: 8.1290x; 1.0025x over previous
import functools

import jax
import jax.numpy as jnp
from jax.experimental import pallas as pl
from jax.experimental.pallas import tpu as pltpu

E = 64
K = 2
H = 768
I = 768
T = 128
ALPHA = 1.702
LIMIT = 7.0
W_LOAD = 0.01
W_IMP = 0.01
W_AUX = 1.0
EPS = 1e-06


EB = 4


def _moe_kernel(x_ref, scale_ref, gw_ref, gb_ref, w1_ref, b1_ref,
                w2_ref, b2_ref, out_ref, aux_ref, t_ref, wfull_ref, pair_ref):
    i = pl.program_id(0)

    @pl.when(i == 0)
    def _prologue():
        xx = x_ref[...]
        ms = jnp.mean(xx * xx, axis=1, keepdims=True)
        t = xx * jax.lax.rsqrt(ms + EPS)
        t = t * scale_ref[...]
        t_ref[...] = t.astype(jnp.bfloat16)
        logits = jax.lax.dot_general(
            t, gw_ref[...], (((1,), (1,)), ((), ())),
            preferred_element_type=jnp.float32) + gb_ref[...]
        lmax = jnp.max(logits, axis=1, keepdims=True)
        ex = jnp.exp(logits - lmax)
        probs = ex / jnp.sum(ex, axis=1, keepdims=True)
        eids = jax.lax.broadcasted_iota(jnp.int32, (T, E), 1)
        m1 = jnp.max(probs, axis=1, keepdims=True)
        a1 = jnp.min(jnp.where(probs == m1, eids, E), axis=1, keepdims=True)
        oh1 = (eids == a1)
        probs2 = jnp.where(oh1, -1.0, probs)
        m2 = jnp.max(probs2, axis=1, keepdims=True)
        a2 = jnp.min(jnp.where(probs2 == m2, eids, E), axis=1, keepdims=True)
        oh2 = (eids == a2)
        denom = m1 + m2
        wfull_ref[...] = (jnp.where(oh1, m1, 0.0) +
                          jnp.where(oh2, m2, 0.0)) / denom
        p_mean = jnp.mean(probs, axis=0)
        c1 = jnp.sum(oh1.astype(jnp.float32), axis=0)
        c2 = jnp.sum(oh2.astype(jnp.float32), axis=0)
        d = (4.0 * c1 + 2.0 * c2) / (T * K)
        loss_lb = W_LOAD * E * jnp.sum(p_mean * d)
        imp = jnp.sum(logits, axis=0)
        imp_mean = jnp.mean(imp)
        var = jnp.sum((imp - imp_mean) ** 2) / (E - 1)
        cv = jnp.sqrt(var) / (imp_mean + 1e-06)
        aux = W_AUX * (loss_lb + W_IMP * cv * cv)
        aux_ref[...] = jnp.reshape(aux, (1, 1))
        out_ref[...] = xx
        j_iota = jax.lax.broadcasted_iota(jnp.int32, (2 * I, I), 0)
        k_iota = jax.lax.broadcasted_iota(jnp.int32, (2 * I, I), 1)
        pair_ref[...] = ((j_iota // 2) == k_iota).astype(jnp.bfloat16)

    tb = t_ref[...]
    eids = jax.lax.broadcasted_iota(jnp.int32, (T, E), 1)
    acc = out_ref[...]
    for p in range(EB):
        e = EB * i + p
        w1 = w1_ref[p].astype(jnp.bfloat16)
        h = jax.lax.dot_general(tb, w1, (((1,), (1,)), ((), ())),
                                preferred_element_type=jnp.float32)
        h = jnp.clip(h + b1_ref[p], -LIMIT, LIMIT)
        lane = jax.lax.broadcasted_iota(jnp.int32, (T, 2 * I), 1)
        evn = (lane & 1) == 0
        s = jax.nn.sigmoid(h * ALPHA)
        gi = h * jnp.where(evn, s, 1.0) + jnp.where(evn, 0.0, 1.0)
        g = jax.lax.dot_general(gi.astype(jnp.bfloat16), pair_ref[...],
                                (((1,), (0,)), ((), ())),
                                preferred_element_type=jnp.float32)
        gb16 = g.astype(jnp.bfloat16)
        w2 = w2_ref[p].astype(jnp.bfloat16)
        o = jax.lax.dot_general(gb16, w2, (((1,), (1,)), ((), ())),
                                preferred_element_type=jnp.float32)
        o = o + b2_ref[p]
        wcol = jnp.sum(jnp.where(eids == e, wfull_ref[...], 0.0), axis=1,
                       keepdims=True)
        acc = acc + wcol * o
    out_ref[...] = acc


@functools.partial(jax.jit)
def kernel(x, norm_scale, gate_w, gate_b, mlp1_weight, mlp1_bias,
           mlp2_weight, mlp2_bias):
    out, aux = pl.pallas_call(
        _moe_kernel,
        grid=(E // EB,),
        in_specs=[
            pl.BlockSpec((T, H), lambda e: (0, 0)),
            pl.BlockSpec((1, H), lambda e: (0, 0)),
            pl.BlockSpec((E, H), lambda e: (0, 0)),
            pl.BlockSpec((1, E), lambda e: (0, 0)),
            pl.BlockSpec((EB, 2 * I, H), lambda e: (e, 0, 0)),
            pl.BlockSpec((EB, 1, 2 * I), lambda e: (e, 0, 0)),
            pl.BlockSpec((EB, H, I), lambda e: (e, 0, 0)),
            pl.BlockSpec((EB, 1, H), lambda e: (e, 0, 0)),
        ],
        out_specs=[
            pl.BlockSpec((T, H), lambda e: (0, 0)),
            pl.BlockSpec((1, 1), lambda e: (0, 0)),
        ],
        out_shape=[
            jax.ShapeDtypeStruct((T, H), jnp.float32),
            jax.ShapeDtypeStruct((1, 1), jnp.float32),
        ],
        scratch_shapes=[
            pltpu.VMEM((T, H), jnp.bfloat16),
            pltpu.VMEM((T, E), jnp.float32),
            pltpu.VMEM((2 * I, I), jnp.bfloat16),
        ],
        compiler_params=pltpu.CompilerParams(
            dimension_semantics=("arbitrary",),
            vmem_limit_bytes=100 * 1024 * 1024),
    )(x, jnp.reshape(norm_scale, (1, H)), gate_w, jnp.reshape(gate_b, (1, E)),
      mlp1_weight, jnp.reshape(mlp1_bias, (E, 1, 2 * I)),
      mlp2_weight, jnp.reshape(mlp2_bias, (E, 1, H)))
    return out, jnp.reshape(aux, ())
